# baseline (speedup 1.0000x reference)
import jax
import jax.numpy as jnp
from jax.experimental import pallas as pl
from jax.experimental.pallas import tpu as pltpu

VOCAB = 1024
E = 256
K = 8
T = 4
H = 4 * E
VB = 1024
NB = VOCAB // VB
QB = 2048


def _layernorm(x, eps=1e-5):
    m = jnp.mean(x, axis=-1, keepdims=True)
    v = jnp.mean((x - m) ** 2, axis=-1, keepdims=True)
    return (x - m) / jnp.sqrt(v + eps)


def _body(emb_ref, z_ref, w1_ref, b1_ref, w2_ref, b2_ref, w3_ref, b3_ref,
          w4_ref, b4_ref, tok_ref, out_ref, mem_ref, q_ref):
    i = pl.program_id(0)

    @pl.when(i < NB)
    def _mlp():
        x = emb_ref[...].reshape(VB * T, E)
        h = jnp.maximum(jnp.dot(x, w1_ref[...]) + b1_ref[...], 0.0)
        h = jnp.maximum(jnp.dot(h, w2_ref[...]) + b2_ref[...], 0.0)
        h = jnp.maximum(jnp.dot(h, w3_ref[...]) + b3_ref[...], 0.0)
        mem = jnp.dot(h, w4_ref[...]) + b4_ref[...]
        mem = _layernorm(mem).reshape(VB, T, E)
        for t in range(T):
            mem_ref[t, pl.ds(i * VB, VB), :] = mem[:, t, :]
        zh = QB // NB
        zi = z_ref[pl.ds(i * zh, zh), :]
        q_ref[pl.ds(i * zh, zh), :] = _layernorm(zi) * (E ** -0.5)

    @pl.when(i >= NB)
    def _attn():
        qr = q_ref[...].reshape(QB // T, T, E)
        outs = []
        for t in range(T):
            qt = qr[:, t, :]
            mt = mem_ref[t]
            s = jax.lax.dot_general(qt, mt, (((1,), (1,)), ((), ())))
            tok_ref[0, t, :] = jnp.argmax(s, axis=-1).astype(jnp.int32)
            p = jnp.exp(s)
            l = jnp.sum(p, axis=-1, keepdims=True)
            outs.append(jnp.dot(p, mt) / l)
        out_ref[...] = jnp.stack(outs, axis=1).reshape(QB, E)


def kernel(z, embeddings, W1, b1, W2, b2, W3, b3, W4, b4):
    n = z.shape[0]
    emb3 = embeddings.reshape(VOCAB, T, E)
    b1r, b2r, b3r = b1.reshape(1, H), b2.reshape(1, H), b3.reshape(1, H)
    b4r = b4.reshape(1, E)
    nq = n // QB

    tok_t, z_q = pl.pallas_call(
        _body,
        grid=(NB + nq,),
        in_specs=[
            pl.BlockSpec((VB, T, E), lambda i: (jnp.minimum(i, NB - 1), 0, 0)),
            pl.BlockSpec((QB, E), lambda i: (jnp.maximum(i - NB, 0), 0)),
            pl.BlockSpec((E, H), lambda i: (0, 0)),
            pl.BlockSpec((1, H), lambda i: (0, 0)),
            pl.BlockSpec((H, H), lambda i: (0, 0)),
            pl.BlockSpec((1, H), lambda i: (0, 0)),
            pl.BlockSpec((H, H), lambda i: (0, 0)),
            pl.BlockSpec((1, H), lambda i: (0, 0)),
            pl.BlockSpec((H, E), lambda i: (0, 0)),
            pl.BlockSpec((1, E), lambda i: (0, 0)),
        ],
        out_specs=[
            pl.BlockSpec((1, T, QB // T), lambda i: (jnp.maximum(i - NB, 0), 0, 0)),
            pl.BlockSpec((QB, E), lambda i: (jnp.maximum(i - NB, 0), 0)),
        ],
        out_shape=[
            jax.ShapeDtypeStruct((nq, T, QB // T), jnp.int32),
            jax.ShapeDtypeStruct((n, E), jnp.float32),
        ],
        scratch_shapes=[
            pltpu.VMEM((T, VOCAB, E), jnp.float32),
            pltpu.VMEM((QB, E), jnp.float32),
        ],
        compiler_params=pltpu.CompilerParams(
            vmem_limit_bytes=100 * 1024 * 1024),
    )(emb3, z, W1, b1r, W2, b2r, W3, b3r, W4, b4r)

    tokens = tok_t.transpose(0, 2, 1).reshape(n)
    return (tokens, z_q)


# speedup vs baseline: 1.0074x; 1.0074x over previous
import jax
import jax.numpy as jnp
from jax.experimental import pallas as pl
from jax.experimental.pallas import tpu as pltpu

VOCAB = 1024
E = 256
K = 8
T = 4
H = 4 * E
VB = 512
NB = VOCAB // VB
QB = 2048


def _layernorm(x, eps=1e-5):
    m = jnp.mean(x, axis=-1, keepdims=True)
    v = jnp.mean((x - m) ** 2, axis=-1, keepdims=True)
    return (x - m) / jnp.sqrt(v + eps)


def _body(emb_ref, z_ref, w1_ref, b1_ref, w2_ref, b2_ref, w3_ref, b3_ref,
          w4_ref, b4_ref, tok_ref, out_ref, mem_ref, q_ref):
    i = pl.program_id(0)

    @pl.when(i < NB)
    def _mlp():
        x = emb_ref[...].reshape(VB * T, E)
        h = jnp.maximum(jnp.dot(x, w1_ref[...]) + b1_ref[...], 0.0)
        h = jnp.maximum(jnp.dot(h, w2_ref[...]) + b2_ref[...], 0.0)
        h = jnp.maximum(jnp.dot(h, w3_ref[...]) + b3_ref[...], 0.0)
        mem = jnp.dot(h, w4_ref[...]) + b4_ref[...]
        mem = _layernorm(mem).reshape(VB, T, E)
        for t in range(T):
            mem_ref[t, pl.ds(i * VB, VB), :] = mem[:, t, :]
        zh = QB // NB
        zi = z_ref[pl.ds(i * zh, zh), :]
        q_ref[pl.ds(i * zh, zh), :] = _layernorm(zi) * (E ** -0.5)

    @pl.when(i >= NB)
    def _attn():
        qr = q_ref[...].reshape(QB // T, T, E)
        outs = []
        for t in range(T):
            qt = qr[:, t, :]
            mt = mem_ref[t]
            s = jax.lax.dot_general(qt, mt, (((1,), (1,)), ((), ())))
            tok_ref[0, t, :] = jnp.argmax(s, axis=-1).astype(jnp.int32)
            p = jnp.exp(s)
            l = jnp.sum(p, axis=-1, keepdims=True)
            outs.append(jnp.dot(p, mt) / l)
        out_ref[...] = jnp.stack(outs, axis=1).reshape(QB, E)


def kernel(z, embeddings, W1, b1, W2, b2, W3, b3, W4, b4):
    n = z.shape[0]
    emb3 = embeddings.reshape(VOCAB, T, E)
    b1r, b2r, b3r = b1.reshape(1, H), b2.reshape(1, H), b3.reshape(1, H)
    b4r = b4.reshape(1, E)
    nq = n // QB

    tok_t, z_q = pl.pallas_call(
        _body,
        grid=(NB + nq,),
        in_specs=[
            pl.BlockSpec((VB, T, E), lambda i: (jnp.minimum(i, NB - 1), 0, 0)),
            pl.BlockSpec((QB, E), lambda i: (jnp.maximum(i - NB, 0), 0)),
            pl.BlockSpec((E, H), lambda i: (0, 0)),
            pl.BlockSpec((1, H), lambda i: (0, 0)),
            pl.BlockSpec((H, H), lambda i: (0, 0)),
            pl.BlockSpec((1, H), lambda i: (0, 0)),
            pl.BlockSpec((H, H), lambda i: (0, 0)),
            pl.BlockSpec((1, H), lambda i: (0, 0)),
            pl.BlockSpec((H, E), lambda i: (0, 0)),
            pl.BlockSpec((1, E), lambda i: (0, 0)),
        ],
        out_specs=[
            pl.BlockSpec((1, T, QB // T), lambda i: (jnp.maximum(i - NB, 0), 0, 0)),
            pl.BlockSpec((QB, E), lambda i: (jnp.maximum(i - NB, 0), 0)),
        ],
        out_shape=[
            jax.ShapeDtypeStruct((nq, T, QB // T), jnp.int32),
            jax.ShapeDtypeStruct((n, E), jnp.float32),
        ],
        scratch_shapes=[
            pltpu.VMEM((T, VOCAB, E), jnp.float32),
            pltpu.VMEM((QB, E), jnp.float32),
        ],
        compiler_params=pltpu.CompilerParams(
            vmem_limit_bytes=100 * 1024 * 1024),
    )(emb3, z, W1, b1r, W2, b2r, W3, b3r, W4, b4r)

    tokens = tok_t.transpose(0, 2, 1).reshape(n)
    return (tokens, z_q)
